# baseline (speedup 1.0000x reference)
import functools

import jax
import jax.numpy as jnp
from jax import lax
from jax.experimental import pallas as pl
from jax.experimental.pallas import tpu as pltpu
from jax.experimental.pallas import tpu_sc as plsc

_NC = 2
_NS = 16
_NW = _NC * _NS

E = 8
H = 1024
I = 2048
T = 2048
K = 2
B = 768
NB = 13
TJ = 1024
J = I // TJ


def _scan_rowmajor(m):
    i128 = lax.broadcasted_iota(jnp.int32, (128, 128), 0)
    j128 = lax.broadcasted_iota(jnp.int32, (128, 128), 1)
    tri_incl = (i128 <= j128).astype(jnp.float32)
    c = lax.dot_general(m, tri_incl, (((1,), (0,)), ((), ())),
                        preferred_element_type=jnp.float32)
    rt = c[:, 127:128]
    i16 = lax.broadcasted_iota(jnp.int32, (16, 16), 0)
    k16 = lax.broadcasted_iota(jnp.int32, (16, 16), 1)
    tri_strict = (k16 < i16).astype(jnp.float32)
    pre = lax.dot_general(tri_strict, rt, (((1,), (0,)), ((), ())),
                          preferred_element_type=jnp.float32)
    return c + pre


def _meta_kernel(e0_ref, e1_ref, row0_ref, row1_ref, be_ref):
    e0 = e0_ref[...]
    e1 = e1_ref[...]
    row0 = jnp.zeros_like(e0)
    row1 = jnp.zeros_like(e1)
    base = jnp.int32(0)
    lastex = jnp.int32(0)
    bstarts = []
    for ex in range(E):
        m0 = (e0 == ex).astype(jnp.float32)
        m1 = (e1 == ex).astype(jnp.float32)
        c0 = _scan_rowmajor(m0)
        ex0 = (c0 - m0).astype(jnp.int32)
        tot0 = jnp.sum(m0).astype(jnp.int32)
        c1 = _scan_rowmajor(m1)
        ex1 = (c1 - m1).astype(jnp.int32) + tot0
        tot = tot0 + jnp.sum(m1).astype(jnp.int32)
        row0 = jnp.where(m0 > 0, base + ex0, row0)
        row1 = jnp.where(m1 > 0, base + ex1, row1)
        bstarts.append(base // B)
        nblk = (tot + B - 1) // B
        lastex = jnp.where(tot > 0, jnp.int32(ex), lastex)
        base = base + nblk * B
    ub = base // B
    barr = lax.broadcasted_iota(jnp.int32, (1, 128), 1)
    be = jnp.zeros((1, 128), jnp.int32)
    for ex in range(E):
        be = be + (barr >= bstarts[ex]).astype(jnp.int32)
    be = be - 1
    be = jnp.where(barr >= ub, lastex, be)
    be = jnp.where(barr == NB, ub, be)
    row0_ref[...] = row0
    row1_ref[...] = row1
    be_ref[...] = be


def _ffn_kernel(be_ref, ub_ref, x_ref, g_ref, u_ref, d_ref, o_ref):
    b = pl.program_id(0)
    j = pl.program_id(1)

    @pl.when(b < ub_ref[0])
    def _():
        x = x_ref[...]
        g = lax.dot_general(x, g_ref[0], (((1,), (1,)), ((), ())),
                            preferred_element_type=jnp.float32)
        u = lax.dot_general(x, u_ref[0], (((1,), (1,)), ((), ())),
                            preferred_element_type=jnp.float32)
        h = (g * jax.nn.sigmoid(g)) * u
        y = lax.dot_general(h, d_ref[0], (((1,), (1,)), ((), ())),
                            preferred_element_type=jnp.float32)

        @pl.when(j == 0)
        def _():
            o_ref[...] = y

        @pl.when(j > 0)
        def _():
            o_ref[...] += y


_TPW = T // _NW


def _dispatch_sc(hidden_states, row0, row1):
    mesh = plsc.VectorSubcoreMesh(core_axis_name="c", subcore_axis_name="s")

    @functools.partial(
        pl.kernel,
        out_type=jax.ShapeDtypeStruct((NB * B, H), jnp.float32),
        mesh=mesh,
        scratch_types=[
            pltpu.VMEM((_TPW, H), jnp.float32),
            pltpu.VMEM((_TPW,), jnp.int32),
            pltpu.VMEM((_TPW,), jnp.int32),
            pltpu.SemaphoreType.DMA,
        ],
    )
    def k(hid_hbm, row0_hbm, row1_hbm, xs_hbm, xbuf, idx0_v, idx1_v, sem):
        wid = lax.axis_index("s") * _NC + lax.axis_index("c")
        base = wid * _TPW
        pltpu.sync_copy(row0_hbm.at[pl.ds(base, _TPW)], idx0_v)
        pltpu.sync_copy(row1_hbm.at[pl.ds(base, _TPW)], idx1_v)
        pltpu.sync_copy(hid_hbm.at[pl.ds(base, _TPW)], xbuf)
        c0 = pltpu.async_copy(xbuf, xs_hbm.at[idx0_v], sem)
        c1 = pltpu.async_copy(xbuf, xs_hbm.at[idx1_v], sem)
        c0.wait()
        c1.wait()

    return k(hidden_states, row0, row1)


_CTOK = 16
_NCH = _TPW // _CTOK


def _combine_sc(y, row0, row1, rw0, rw1):
    mesh = plsc.VectorSubcoreMesh(core_axis_name="c", subcore_axis_name="s")

    @functools.partial(
        pl.kernel,
        out_type=jax.ShapeDtypeStruct((T, H), jnp.float32),
        mesh=mesh,
        scratch_types=[
            pltpu.VMEM((2, _CTOK, H), jnp.float32),
            pltpu.VMEM((2, _CTOK, H), jnp.float32),
            pltpu.VMEM((2, _CTOK), jnp.int32),
            pltpu.VMEM((2, _CTOK), jnp.int32),
            pltpu.VMEM((_CTOK, 16), jnp.float32),
            pltpu.VMEM((_CTOK, 16), jnp.float32),
            pltpu.SemaphoreType.DMA,
            pltpu.SemaphoreType.DMA,
        ],
    )
    def k(y_hbm, row0_hbm, row1_hbm, rw0_hbm, rw1_hbm, out_hbm,
          buf0, buf1, idx0_v, idx1_v, w0_v, w1_v, sem0, sem1):
        wid = lax.axis_index("s") * _NC + lax.axis_index("c")
        sems = (sem0, sem1)

        def start(g):
            s = g % 2
            base = wid * _TPW + g * _CTOK
            pltpu.sync_copy(row0_hbm.at[pl.ds(base, _CTOK)], idx0_v.at[s])
            pltpu.sync_copy(row1_hbm.at[pl.ds(base, _CTOK)], idx1_v.at[s])
            pltpu.async_copy(y_hbm.at[idx0_v.at[s]], buf0.at[s], sems[s])
            pltpu.async_copy(y_hbm.at[idx1_v.at[s]], buf1.at[s], sems[s])

        def finish(g):
            s = g % 2
            base = wid * _TPW + g * _CTOK
            pltpu.make_async_copy(y_hbm.at[idx0_v.at[s]], buf0.at[s],
                                  sems[s]).wait()
            pltpu.make_async_copy(y_hbm.at[idx1_v.at[s]], buf1.at[s],
                                  sems[s]).wait()
            pltpu.sync_copy(rw0_hbm.at[pl.ds(base, _CTOK)], w0_v)
            pltpu.sync_copy(rw1_hbm.at[pl.ds(base, _CTOK)], w1_v)

            def tok(t, _):
                w0 = w0_v[t, :]
                w1 = w1_v[t, :]
                for c in range(H // 16):
                    sl = pl.ds(c * 16, 16)
                    buf0[s, t, sl] = w0 * buf0[s, t, sl] + w1 * buf1[s, t, sl]
                return 0

            lax.fori_loop(0, _CTOK, tok, 0)
            pltpu.sync_copy(buf0.at[s], out_hbm.at[pl.ds(base, _CTOK)])

        start(0)
        for g in range(_NCH):
            if g + 1 < _NCH:
                start(g + 1)
            finish(g)

    return k(y, row0, row1, rw0, rw1)


def _run_meta(e0, e1, interpret=False):
    return pl.pallas_call(
        _meta_kernel,
        out_shape=(
            jax.ShapeDtypeStruct((16, 128), jnp.int32),
            jax.ShapeDtypeStruct((16, 128), jnp.int32),
            jax.ShapeDtypeStruct((1, 128), jnp.int32),
        ),
        interpret=interpret,
    )(e0, e1)


def _run_ffn(be, ub, x_s, gate_up_proj, down_proj, interpret=False):
    grid_spec = pltpu.PrefetchScalarGridSpec(
        num_scalar_prefetch=2,
        grid=(NB, J),
        in_specs=[
            pl.BlockSpec((B, H), lambda b, j, be, ub: (b, 0)),
            pl.BlockSpec((1, TJ, H), lambda b, j, be, ub: (be[b], j, 0)),
            pl.BlockSpec((1, TJ, H), lambda b, j, be, ub: (be[b], j + J, 0)),
            pl.BlockSpec((1, H, TJ), lambda b, j, be, ub: (be[b], 0, j)),
        ],
        out_specs=pl.BlockSpec((B, H), lambda b, j, be, ub: (b, 0)),
    )
    return pl.pallas_call(
        _ffn_kernel,
        grid_spec=grid_spec,
        out_shape=jax.ShapeDtypeStruct((NB * B, H), jnp.float32),
        interpret=interpret,
    )(be, ub, x_s, gate_up_proj, gate_up_proj, down_proj)


def _impl(hidden_states, selected_experts, router_weights, gate_up_proj,
          down_proj, interpret=False):
    sel = selected_experts.astype(jnp.int32)
    e0 = sel[:, 0].reshape(16, 128)
    e1 = sel[:, 1].reshape(16, 128)
    row0, row1, bemix = _run_meta(e0, e1, interpret=interpret)
    row0 = row0.reshape(T)
    row1 = row1.reshape(T)
    be = bemix[0, :NB]
    ub = bemix[0, NB:NB + 1]

    if interpret:
        x_s = jnp.zeros((NB * B, H), jnp.float32)
        x_s = x_s.at[row0].set(hidden_states)
        x_s = x_s.at[row1].set(hidden_states)
        y = _run_ffn(be, ub, x_s, gate_up_proj, down_proj, interpret=True)
        w0 = router_weights[:, 0:1]
        w1 = router_weights[:, 1:2]
        return w0 * y[row0] + w1 * y[row1]

    x_s = _dispatch_sc(hidden_states, row0, row1)
    y = _run_ffn(be, ub, x_s, gate_up_proj, down_proj)
    rw0 = jnp.broadcast_to(router_weights[:, 0:1], (T, 16))
    rw1 = jnp.broadcast_to(router_weights[:, 1:2], (T, 16))
    return _combine_sc(y, row0, row1, rw0, rw1)


def kernel(hidden_states, selected_experts, router_weights, gate_up_proj,
           down_proj):
    return _impl(hidden_states, selected_experts, router_weights,
                 gate_up_proj, down_proj)


# speedup vs baseline: 1.0391x; 1.0391x over previous
import functools

import jax
import jax.numpy as jnp
from jax import lax
from jax.experimental import pallas as pl
from jax.experimental.pallas import tpu as pltpu
from jax.experimental.pallas import tpu_sc as plsc

_NC = 2
_NS = 16
_NW = _NC * _NS

E = 8
H = 1024
I = 2048
T = 2048
K = 2
B = 768
NB = 13
TJ = 1024
J = I // TJ


def _scan_rowmajor(m):
    i128 = lax.broadcasted_iota(jnp.int32, (128, 128), 0)
    j128 = lax.broadcasted_iota(jnp.int32, (128, 128), 1)
    tri_incl = (i128 <= j128).astype(jnp.float32)
    c = lax.dot_general(m, tri_incl, (((1,), (0,)), ((), ())),
                        preferred_element_type=jnp.float32)
    rt = c[:, 127:128]
    i16 = lax.broadcasted_iota(jnp.int32, (16, 16), 0)
    k16 = lax.broadcasted_iota(jnp.int32, (16, 16), 1)
    tri_strict = (k16 < i16).astype(jnp.float32)
    pre = lax.dot_general(tri_strict, rt, (((1,), (0,)), ((), ())),
                          preferred_element_type=jnp.float32)
    return c + pre


def _meta_kernel(e0_ref, e1_ref, row0_ref, row1_ref, be_ref):
    e0 = e0_ref[...]
    e1 = e1_ref[...]
    row0 = jnp.zeros_like(e0)
    row1 = jnp.zeros_like(e1)
    base = jnp.int32(0)
    lastex = jnp.int32(0)
    bstarts = []
    for ex in range(E):
        m0 = (e0 == ex).astype(jnp.float32)
        m1 = (e1 == ex).astype(jnp.float32)
        c0 = _scan_rowmajor(m0)
        ex0 = (c0 - m0).astype(jnp.int32)
        tot0 = jnp.sum(m0).astype(jnp.int32)
        c1 = _scan_rowmajor(m1)
        ex1 = (c1 - m1).astype(jnp.int32) + tot0
        tot = tot0 + jnp.sum(m1).astype(jnp.int32)
        row0 = jnp.where(m0 > 0, base + ex0, row0)
        row1 = jnp.where(m1 > 0, base + ex1, row1)
        bstarts.append(base // B)
        nblk = (tot + B - 1) // B
        lastex = jnp.where(tot > 0, jnp.int32(ex), lastex)
        base = base + nblk * B
    ub = base // B
    barr = lax.broadcasted_iota(jnp.int32, (1, 128), 1)
    be = jnp.zeros((1, 128), jnp.int32)
    for ex in range(E):
        be = be + (barr >= bstarts[ex]).astype(jnp.int32)
    be = be - 1
    be = jnp.where(barr >= ub, lastex, be)
    be = jnp.where(barr == NB, ub, be)
    row0_ref[...] = row0
    row1_ref[...] = row1
    be_ref[...] = be


def _ffn_kernel(be_ref, ub_ref, x_ref, g_ref, u_ref, d_ref, o_ref):
    b = pl.program_id(0)
    j = pl.program_id(1)

    @pl.when(b < ub_ref[0])
    def _():
        x = x_ref[...]
        g = lax.dot_general(x, g_ref[0], (((1,), (1,)), ((), ())),
                            preferred_element_type=jnp.float32)
        u = lax.dot_general(x, u_ref[0], (((1,), (1,)), ((), ())),
                            preferred_element_type=jnp.float32)
        h = (g * jax.nn.sigmoid(g)) * u
        y = lax.dot_general(h, d_ref[0], (((1,), (1,)), ((), ())),
                            preferred_element_type=jnp.float32)

        @pl.when(j == 0)
        def _():
            o_ref[...] = y

        @pl.when(j > 0)
        def _():
            o_ref[...] += y


_TPW = T // _NW


def _dispatch_sc(hidden_states, row0, row1):
    mesh = plsc.VectorSubcoreMesh(core_axis_name="c", subcore_axis_name="s")

    @functools.partial(
        pl.kernel,
        out_type=jax.ShapeDtypeStruct((NB * B, H), jnp.float32),
        mesh=mesh,
        scratch_types=[
            pltpu.VMEM((_TPW, H), jnp.float32),
            pltpu.VMEM((_TPW,), jnp.int32),
            pltpu.VMEM((_TPW,), jnp.int32),
            pltpu.SemaphoreType.DMA,
        ],
    )
    def k(hid_hbm, row0_hbm, row1_hbm, xs_hbm, xbuf, idx0_v, idx1_v, sem):
        wid = lax.axis_index("s") * _NC + lax.axis_index("c")
        base = wid * _TPW
        pltpu.sync_copy(row0_hbm.at[pl.ds(base, _TPW)], idx0_v)
        pltpu.sync_copy(row1_hbm.at[pl.ds(base, _TPW)], idx1_v)
        pltpu.sync_copy(hid_hbm.at[pl.ds(base, _TPW)], xbuf)
        c0 = pltpu.async_copy(xbuf, xs_hbm.at[idx0_v], sem)
        c1 = pltpu.async_copy(xbuf, xs_hbm.at[idx1_v], sem)
        c0.wait()
        c1.wait()

    return k(hidden_states, row0, row1)


_CTOK = 32


def _combine_sc(y, row0, row1, rw0, rw1):
    mesh = plsc.VectorSubcoreMesh(core_axis_name="c", subcore_axis_name="s")

    @functools.partial(
        pl.kernel,
        out_type=jax.ShapeDtypeStruct((T, H), jnp.float32),
        mesh=mesh,
        scratch_types=[
            pltpu.VMEM((_CTOK, H), jnp.float32),
            pltpu.VMEM((_CTOK, H), jnp.float32),
            pltpu.VMEM((_CTOK,), jnp.int32),
            pltpu.VMEM((_CTOK,), jnp.int32),
            pltpu.VMEM((_CTOK, 16), jnp.float32),
            pltpu.VMEM((_CTOK, 16), jnp.float32),
            pltpu.SemaphoreType.DMA,
            pltpu.SemaphoreType.DMA,
        ],
    )
    def k(y_hbm, row0_hbm, row1_hbm, rw0_hbm, rw1_hbm, out_hbm,
          buf0, buf1, idx0_v, idx1_v, w0_v, w1_v, sem0, sem1):
        wid = lax.axis_index("s") * _NC + lax.axis_index("c")

        def chunk(g, _):
            base = wid * _TPW + g * _CTOK
            pltpu.sync_copy(row0_hbm.at[pl.ds(base, _CTOK)], idx0_v)
            pltpu.sync_copy(row1_hbm.at[pl.ds(base, _CTOK)], idx1_v)
            c0 = pltpu.async_copy(y_hbm.at[idx0_v], buf0, sem0)
            c1 = pltpu.async_copy(y_hbm.at[idx1_v], buf1, sem1)
            pltpu.sync_copy(rw0_hbm.at[pl.ds(base, _CTOK)], w0_v)
            pltpu.sync_copy(rw1_hbm.at[pl.ds(base, _CTOK)], w1_v)
            c0.wait()
            c1.wait()

            def tok(t, _):
                w0 = w0_v[t, :]
                w1 = w1_v[t, :]
                for c in range(H // 16):
                    sl = pl.ds(c * 16, 16)
                    buf0[t, sl] = w0 * buf0[t, sl] + w1 * buf1[t, sl]
                return 0

            lax.fori_loop(0, _CTOK, tok, 0)
            pltpu.sync_copy(buf0, out_hbm.at[pl.ds(base, _CTOK)])
            return 0

        lax.fori_loop(0, _TPW // _CTOK, chunk, 0)

    return k(y, row0, row1, rw0, rw1)


def _run_meta(e0, e1, interpret=False):
    return pl.pallas_call(
        _meta_kernel,
        out_shape=(
            jax.ShapeDtypeStruct((16, 128), jnp.int32),
            jax.ShapeDtypeStruct((16, 128), jnp.int32),
            jax.ShapeDtypeStruct((1, 128), jnp.int32),
        ),
        interpret=interpret,
    )(e0, e1)


def _run_ffn(be, ub, x_s, gate_up_proj, down_proj, interpret=False):
    grid_spec = pltpu.PrefetchScalarGridSpec(
        num_scalar_prefetch=2,
        grid=(NB, J),
        in_specs=[
            pl.BlockSpec((B, H), lambda b, j, be, ub: (b, 0)),
            pl.BlockSpec((1, TJ, H), lambda b, j, be, ub: (be[b], j, 0)),
            pl.BlockSpec((1, TJ, H), lambda b, j, be, ub: (be[b], j + J, 0)),
            pl.BlockSpec((1, H, TJ), lambda b, j, be, ub: (be[b], 0, j)),
        ],
        out_specs=pl.BlockSpec((B, H), lambda b, j, be, ub: (b, 0)),
    )
    return pl.pallas_call(
        _ffn_kernel,
        grid_spec=grid_spec,
        out_shape=jax.ShapeDtypeStruct((NB * B, H), jnp.float32),
        interpret=interpret,
    )(be, ub, x_s, gate_up_proj, gate_up_proj, down_proj)


def _impl(hidden_states, selected_experts, router_weights, gate_up_proj,
          down_proj, interpret=False):
    sel = selected_experts.astype(jnp.int32)
    e0 = sel[:, 0].reshape(16, 128)
    e1 = sel[:, 1].reshape(16, 128)
    row0, row1, bemix = _run_meta(e0, e1, interpret=interpret)
    row0 = row0.reshape(T)
    row1 = row1.reshape(T)
    be = bemix[0, :NB]
    ub = bemix[0, NB:NB + 1]

    if interpret:
        x_s = jnp.zeros((NB * B, H), jnp.float32)
        x_s = x_s.at[row0].set(hidden_states)
        x_s = x_s.at[row1].set(hidden_states)
        y = _run_ffn(be, ub, x_s, gate_up_proj, down_proj, interpret=True)
        w0 = router_weights[:, 0:1]
        w1 = router_weights[:, 1:2]
        return w0 * y[row0] + w1 * y[row1]

    x_s = _dispatch_sc(hidden_states, row0, row1)
    y = _run_ffn(be, ub, x_s, gate_up_proj, down_proj)
    rw0 = jnp.broadcast_to(router_weights[:, 0:1], (T, 16))
    rw1 = jnp.broadcast_to(router_weights[:, 1:2], (T, 16))
    return _combine_sc(y, row0, row1, rw0, rw1)


def kernel(hidden_states, selected_experts, router_weights, gate_up_proj,
           down_proj):
    return _impl(hidden_states, selected_experts, router_weights,
                 gate_up_proj, down_proj)
